# baseline (speedup 1.0000x reference)
import functools

import jax
import jax.numpy as jnp
from jax.experimental import pallas as pl
from jax.experimental.pallas import tpu as pltpu


def _vq_body(z_hbm, cbn_ref, cb_ref, csq_ref, q_hbm, loss_ref,
             zbuf, qbuf, in_sem, out_sem, *, n_groups, k_dim):
    n = pl.program_id(0)
    g = pl.program_id(1)
    n_b = pl.num_programs(0)
    rows = zbuf.shape[1]
    t0 = zbuf.shape[3]
    t = rows * t0 // k_dim
    total = n_b * n_groups
    s = n * n_groups + g
    slot = jax.lax.rem(s, 2)
    nslot = jax.lax.rem(s + 1, 2)

    def in_copy(step, b):
        nn = step // n_groups
        gg = jax.lax.rem(step, n_groups)
        return pltpu.make_async_copy(
            z_hbm.at[pl.ds(gg * rows, rows), pl.ds(nn, 1), :],
            zbuf.at[b], in_sem.at[b])

    def out_copy(step, b):
        nn = step // n_groups
        gg = jax.lax.rem(step, n_groups)
        return pltpu.make_async_copy(
            qbuf.at[b],
            q_hbm.at[pl.ds(gg * rows, rows), pl.ds(nn, 1), :], out_sem.at[b])

    @pl.when(s == 0)
    def _first():
        loss_ref[0, 0] = 0.0
        in_copy(0, 0).start()

    @pl.when(s + 1 < total)
    def _prefetch():
        in_copy(s + 1, nslot).start()

    in_copy(s, slot).wait()

    zb = zbuf[slot].reshape(k_dim, t)
    cbn = cbn_ref[0]
    mm = jax.lax.dot_general(cbn, zb, (((1,), (0,)), ((), ())),
                             preferred_element_type=jnp.float32)
    dist = mm + csq_ref[0]
    idx = jnp.argmin(dist, axis=0)
    onehot = (jax.lax.broadcasted_iota(jnp.int32, dist.shape, 0)
              == idx[None, :]).astype(jnp.bfloat16)
    q = jax.lax.dot_general(cb_ref[0], onehot, (((0,), (0,)), ((), ())),
                            preferred_element_type=jnp.float32)

    @pl.when(s >= 2)
    def _drain_prev():
        out_copy(s - 2, slot).wait()

    qbuf[slot] = q.reshape(rows, 1, t0)
    out_copy(s, slot).start()

    @pl.when(g == n_groups - 1)
    def _acc():
        r = zb - q
        loss_ref[0, 0] += jnp.sum(r * r)

    @pl.when(s == total - 1)
    def _drain_all():
        @pl.when(s >= 1)
        def _():
            out_copy(s - 1, nslot).wait()
        out_copy(s, slot).wait()


def kernel(z, codebooks):
    N, C, T0 = z.shape
    G, CB, K = codebooks.shape
    T = (C * T0) // (G * K)
    rows = C // G

    zt = jnp.transpose(z, (1, 0, 2))

    cb_neg2 = -2.0 * codebooks
    csq = jnp.sum(codebooks * codebooks, axis=2, keepdims=True)

    qt, loss_sum = pl.pallas_call(
        functools.partial(_vq_body, n_groups=G, k_dim=K),
        grid=(N, G),
        in_specs=[
            pl.BlockSpec(memory_space=pltpu.MemorySpace.HBM),
            pl.BlockSpec((1, CB, K), lambda n, g: (g, 0, 0)),
            pl.BlockSpec((1, CB, K), lambda n, g: (g, 0, 0)),
            pl.BlockSpec((1, CB, 1), lambda n, g: (g, 0, 0)),
        ],
        out_specs=[
            pl.BlockSpec(memory_space=pltpu.MemorySpace.HBM),
            pl.BlockSpec(memory_space=pltpu.SMEM),
        ],
        out_shape=[
            jax.ShapeDtypeStruct((C, N, T0), jnp.float32),
            jax.ShapeDtypeStruct((1, 1), jnp.float32),
        ],
        scratch_shapes=[
            pltpu.VMEM((2, rows, 1, T0), jnp.float32),
            pltpu.VMEM((2, rows, 1, T0), jnp.float32),
            pltpu.SemaphoreType.DMA((2,)),
            pltpu.SemaphoreType.DMA((2,)),
        ],
    )(zt, cb_neg2, codebooks.astype(jnp.bfloat16), csq)

    q = jnp.transpose(qt, (1, 0, 2))
    vq_loss = loss_sum[0, 0] * (0.25 / (N * K * T))
    return q, vq_loss


# speedup vs baseline: 1.0162x; 1.0162x over previous
import functools

import jax
import jax.numpy as jnp
from jax.experimental import pallas as pl
from jax.experimental.pallas import tpu as pltpu


def _vq_body(z_hbm, cbn_ref, cb_ref, csq_ref, q_hbm, loss_ref,
             zbuf, qbuf, in_sem, out_sem, *, n_groups, k_dim):
    n = pl.program_id(0)
    g = pl.program_id(1)
    n_b = pl.num_programs(0)
    rows = zbuf.shape[1]
    t0 = zbuf.shape[3]
    t = rows * t0 // k_dim
    total = n_b * n_groups
    s = n * n_groups + g
    slot = jax.lax.rem(s, 2)
    nslot = jax.lax.rem(s + 1, 2)

    def in_copy(step, b):
        nn = step // n_groups
        gg = jax.lax.rem(step, n_groups)
        return pltpu.make_async_copy(
            z_hbm.at[pl.ds(gg * rows, rows), pl.ds(nn, 1), :],
            zbuf.at[b], in_sem.at[b])

    def out_copy(step, b):
        nn = step // n_groups
        gg = jax.lax.rem(step, n_groups)
        return pltpu.make_async_copy(
            qbuf.at[b],
            q_hbm.at[pl.ds(gg * rows, rows), pl.ds(nn, 1), :], out_sem.at[b])

    @pl.when(s == 0)
    def _first():
        loss_ref[0, 0] = 0.0
        in_copy(0, 0).start()

    @pl.when(s + 1 < total)
    def _prefetch():
        in_copy(s + 1, nslot).start()

    in_copy(s, slot).wait()

    zb = zbuf[slot].reshape(k_dim, t)
    cbn = cbn_ref[0]
    mm = jax.lax.dot_general(cbn, zb, (((1,), (0,)), ((), ())),
                             preferred_element_type=jnp.float32)
    dist = mm + csq_ref[0]
    idx = jnp.argmin(dist, axis=0)
    onehot = (jax.lax.broadcasted_iota(jnp.int32, dist.shape, 0)
              == idx[None, :]).astype(jnp.float32)
    q = jax.lax.dot_general(cb_ref[0], onehot, (((0,), (0,)), ((), ())),
                            preferred_element_type=jnp.float32)

    @pl.when(s >= 2)
    def _drain_prev():
        out_copy(s - 2, slot).wait()

    qbuf[slot] = q.reshape(rows, 1, t0)
    out_copy(s, slot).start()

    @pl.when(g == n_groups - 1)
    def _acc():
        r = zb - q
        loss_ref[0, 0] += jnp.sum(r * r)

    @pl.when(s == total - 1)
    def _drain_all():
        @pl.when(s >= 1)
        def _():
            out_copy(s - 1, nslot).wait()
        out_copy(s, slot).wait()


def kernel(z, codebooks):
    N, C, T0 = z.shape
    G, CB, K = codebooks.shape
    T = (C * T0) // (G * K)
    rows = C // G

    zt = jnp.transpose(z, (1, 0, 2))

    cb_neg2 = -2.0 * codebooks
    csq = jnp.sum(codebooks * codebooks, axis=2, keepdims=True)

    qt, loss_sum = pl.pallas_call(
        functools.partial(_vq_body, n_groups=G, k_dim=K),
        grid=(N, G),
        in_specs=[
            pl.BlockSpec(memory_space=pltpu.MemorySpace.HBM),
            pl.BlockSpec((1, CB, K), lambda n, g: (g, 0, 0)),
            pl.BlockSpec((1, CB, K), lambda n, g: (g, 0, 0)),
            pl.BlockSpec((1, CB, 1), lambda n, g: (g, 0, 0)),
        ],
        out_specs=[
            pl.BlockSpec(memory_space=pltpu.MemorySpace.HBM),
            pl.BlockSpec(memory_space=pltpu.SMEM),
        ],
        out_shape=[
            jax.ShapeDtypeStruct((C, N, T0), jnp.float32),
            jax.ShapeDtypeStruct((1, 1), jnp.float32),
        ],
        scratch_shapes=[
            pltpu.VMEM((2, rows, 1, T0), jnp.float32),
            pltpu.VMEM((2, rows, 1, T0), jnp.float32),
            pltpu.SemaphoreType.DMA((2,)),
            pltpu.SemaphoreType.DMA((2,)),
        ],
    )(zt, cb_neg2, codebooks, csq)

    q = jnp.transpose(qt, (1, 0, 2))
    vq_loss = loss_sum[0, 0] * (0.25 / (N * K * T))
    return q, vq_loss


# speedup vs baseline: 1.0206x; 1.0043x over previous
import functools

import jax
import jax.numpy as jnp
from jax.experimental import pallas as pl
from jax.experimental.pallas import tpu as pltpu


def _vq_body(z_hbm, cbn_ref, cb_ref, csq_ref, q_hbm, loss_ref,
             zbuf, qbuf, in_sem, out_sem, *, n_groups, k_dim):
    g = pl.program_id(0)
    n = pl.program_id(1)
    n_b = pl.num_programs(1)
    rows = zbuf.shape[1]
    t0 = zbuf.shape[3]
    t = rows * t0 // k_dim
    total = n_b * n_groups
    s = g * n_b + n
    slot = jax.lax.rem(s, 2)
    nslot = jax.lax.rem(s + 1, 2)

    def in_copy(step, b):
        gg = step // n_b
        nn = jax.lax.rem(step, n_b)
        return pltpu.make_async_copy(
            z_hbm.at[pl.ds(gg * rows, rows), pl.ds(nn, 1), :],
            zbuf.at[b], in_sem.at[b])

    def out_copy(step, b):
        gg = step // n_b
        nn = jax.lax.rem(step, n_b)
        return pltpu.make_async_copy(
            qbuf.at[b],
            q_hbm.at[pl.ds(gg * rows, rows), pl.ds(nn, 1), :], out_sem.at[b])

    @pl.when(s == 0)
    def _first():
        loss_ref[0, 0] = 0.0
        in_copy(0, 0).start()

    @pl.when(s + 1 < total)
    def _prefetch():
        in_copy(s + 1, nslot).start()

    in_copy(s, slot).wait()

    zb = zbuf[slot].reshape(k_dim, t)
    cbn = cbn_ref[0]
    mm = jax.lax.dot_general(cbn, zb, (((1,), (0,)), ((), ())),
                             preferred_element_type=jnp.float32)
    dist = mm + csq_ref[0]
    idx = jnp.argmin(dist, axis=0)
    onehot = (jax.lax.broadcasted_iota(jnp.int32, dist.shape, 0)
              == idx[None, :]).astype(jnp.float32)
    q = jax.lax.dot_general(cb_ref[0], onehot, (((0,), (0,)), ((), ())),
                            preferred_element_type=jnp.float32)

    @pl.when(s >= 2)
    def _drain_prev():
        out_copy(s - 2, slot).wait()

    qbuf[slot] = q.reshape(rows, 1, t0)
    out_copy(s, slot).start()

    @pl.when(g == n_groups - 1)
    def _acc():
        r = zb - q
        loss_ref[0, 0] += jnp.sum(r * r)

    @pl.when(s == total - 1)
    def _drain_all():
        @pl.when(s >= 1)
        def _():
            out_copy(s - 1, nslot).wait()
        out_copy(s, slot).wait()


def kernel(z, codebooks):
    N, C, T0 = z.shape
    G, CB, K = codebooks.shape
    T = (C * T0) // (G * K)
    rows = C // G

    zt = jnp.transpose(z, (1, 0, 2))

    cb_neg2 = -2.0 * codebooks
    csq = jnp.sum(codebooks * codebooks, axis=2, keepdims=True)

    qt, loss_sum = pl.pallas_call(
        functools.partial(_vq_body, n_groups=G, k_dim=K),
        grid=(G, N),
        in_specs=[
            pl.BlockSpec(memory_space=pltpu.MemorySpace.HBM),
            pl.BlockSpec((1, CB, K), lambda g, n: (g, 0, 0)),
            pl.BlockSpec((1, CB, K), lambda g, n: (g, 0, 0)),
            pl.BlockSpec((1, CB, 1), lambda g, n: (g, 0, 0)),
        ],
        out_specs=[
            pl.BlockSpec(memory_space=pltpu.MemorySpace.HBM),
            pl.BlockSpec(memory_space=pltpu.SMEM),
        ],
        out_shape=[
            jax.ShapeDtypeStruct((C, N, T0), jnp.float32),
            jax.ShapeDtypeStruct((1, 1), jnp.float32),
        ],
        scratch_shapes=[
            pltpu.VMEM((2, rows, 1, T0), jnp.float32),
            pltpu.VMEM((2, rows, 1, T0), jnp.float32),
            pltpu.SemaphoreType.DMA((2,)),
            pltpu.SemaphoreType.DMA((2,)),
        ],
    )(zt, cb_neg2, codebooks, csq)

    q = jnp.transpose(qt, (1, 0, 2))
    vq_loss = loss_sum[0, 0] * (0.25 / (N * K * T))
    return q, vq_loss
